# baseline (speedup 1.0000x reference)
import jax
import jax.numpy as jnp
from jax import lax
from jax.experimental import pallas as pl

ALPHA = 0.2
N_NODE = 10000
N_EDGE = 1024
D = 128

TE2 = 256
TE3 = 128

f32 = jnp.float32
bf16 = jnp.bfloat16


def _lrelu(t):
    return jnp.where(t > 0, t, ALPHA * t)


def _s_col(xa_ref, a_ref, wc_ref):
    c = jnp.dot(wc_ref[...], a_ref[0:D, :], preferred_element_type=f32)
    s = jnp.dot(xa_ref[...], a_ref[D:2 * D, :], preferred_element_type=f32)
    return _lrelu(c[0, 0] + s)


def _q_col(xa_ref, a2_ref):
    return jnp.dot(xa_ref[...], a2_ref[0:D, :], preferred_element_type=f32)


def _aug_ones(m, t):
    return jnp.concatenate([m.astype(bf16), jnp.ones((t, D), bf16)], axis=1)


def _cast_body(ht_ref, hbt_ref):
    hbt_ref[...] = ht_ref[...].astype(bf16)


def _prep_body(x_ref, w2_ref, a_ref, a2_ref, wc_ref,
               xa_ref, xvb_ref, smax_ref, qmax_ref):
    x = x_ref[...]
    xa = jnp.dot(x, w2_ref[...], preferred_element_type=f32)
    xa_ref[...] = xa
    xvb_ref[...] = _aug_ones(x, N_NODE)
    smax_ref[...] = jnp.max(_s_col(xa_ref, a_ref, wc_ref)).reshape(1, 1)
    qmax_ref[...] = jnp.max(_q_col(xa_ref, a2_ref)).reshape(1, 1)


def _edge_body(hbt_ref, xa_ref, a_ref, wc_ref, smax_ref, xvb_ref, w3_ref,
               a2_ref, edge_ref, eaug_ref, y_ref, ymax_ref):
    @pl.when(pl.program_id(0) == 0)
    def _():
        ymax_ref[...] = jnp.full((1, 1), -jnp.inf, f32)

    s = _s_col(xa_ref, a_ref, wc_ref)
    expw = jnp.exp(s - smax_ref[0, 0]).astype(bf16)
    w = hbt_ref[...] * expw
    numaug = lax.dot_general(w, xvb_ref[...], (((0,), (0,)), ((), ())),
                             preferred_element_type=f32)
    num = numaug[:, :D]
    den = numaug[:, D:D + 1]
    mx = jnp.sum(xvb_ref[...][:, :D].astype(f32), axis=0, keepdims=True) \
        * (1.0 / N_NODE)
    edge = jnp.where(den > 0, num / den, mx)
    edge_ref[...] = edge
    eaug_ref[...] = _aug_ones(edge, TE2)
    w3a = jnp.dot(w3_ref[...], a2_ref[D:2 * D, :],
                  preferred_element_type=f32)
    y = lax.dot_general(w3a, edge, (((0,), (1,)), ((), ())),
                        preferred_element_type=f32)
    y_ref[...] = y
    ymax_ref[...] = jnp.maximum(ymax_ref[...], jnp.max(y).reshape(1, 1))


def _node_body(hbt_ref, xa_ref, a2_ref, qmax_ref, y_ref, ymax_ref, eaug_ref,
               out_ref):
    @pl.when(pl.program_id(0) == 0)
    def _():
        out_ref[...] = jnp.zeros_like(out_ref)

    q = _q_col(xa_ref, a2_ref)
    qm = qmax_ref[0, 0]
    ym = ymax_ref[0, 0]
    acol = jnp.exp(q - qm).astype(bf16)
    ccol = jnp.exp(ALPHA * q - qm).astype(bf16)
    y = y_ref[...]
    brow = jnp.exp(y - ym).astype(bf16)
    drow = jnp.exp(ALPHA * y - ym).astype(bf16)
    w = hbt_ref[...] * jnp.maximum(acol * brow, ccol * drow)
    out_ref[...] += lax.dot_general(w, eaug_ref[...], (((1,), (0,)), ((), ())),
                                    preferred_element_type=f32)


def _norm_prep_body(aug_ref, edge_ref, w2_ref, w_ref, a_ref, a2_ref, wc_ref,
                    xa_ref, xvb_ref, smax_ref, qmax_ref):
    aug = aug_ref[...]
    num = aug[:, :D]
    den = aug[:, D:D + 1]
    emean = jnp.sum(edge_ref[...], axis=0, keepdims=True) * (1.0 / N_EDGE)
    node = jnp.where(den > 0, num / den, emean)
    h = jnp.where(node > 0, node, jnp.exp(node) - 1.0)
    xa = jnp.dot(h, w2_ref[...], preferred_element_type=f32)
    xa_ref[...] = xa
    xv = jnp.dot(h, w_ref[...], preferred_element_type=f32)
    xvb_ref[...] = _aug_ones(xv, N_NODE)
    smax_ref[...] = jnp.max(_s_col(xa_ref, a_ref, wc_ref)).reshape(1, 1)
    qmax_ref[...] = jnp.max(_q_col(xa_ref, a2_ref)).reshape(1, 1)


def _norm_body(aug_ref, edge_ref, out_ref):
    aug = aug_ref[...]
    num = aug[:, :D]
    den = aug[:, D:D + 1]
    emean = jnp.sum(edge_ref[...], axis=0, keepdims=True) * (1.0 / N_EDGE)
    out_ref[...] = jnp.where(den > 0, num / den, emean)


def _full(shape):
    nd = len(shape)
    return pl.BlockSpec(shape, lambda i: (0,) * nd)


def _rows(t):
    nd = len(t)
    return pl.BlockSpec(t, lambda i: (i,) + (0,) * (nd - 1))


def _cols(t):
    return pl.BlockSpec(t, lambda i: (0, i))


def _edge_pass(Hbt, xa, a, wc_r, smax, xvb, w3, a2):
    n, e, d = N_NODE, N_EDGE, D
    return pl.pallas_call(
        _edge_body,
        grid=(e // TE2,),
        in_specs=[_cols((n, TE2)), _full((n, d)), _full((2 * d, 1)),
                  _full((1, d)), _full((1, 1)), _full((n, 2 * d)),
                  _full((d, d)), _full((2 * d, 1))],
        out_specs=[_rows((TE2, d)), _rows((TE2, 2 * d)),
                   _cols((1, TE2)), _full((1, 1))],
        out_shape=[jax.ShapeDtypeStruct((e, d), f32),
                   jax.ShapeDtypeStruct((e, 2 * d), bf16),
                   jax.ShapeDtypeStruct((1, e), f32),
                   jax.ShapeDtypeStruct((1, 1), f32)],
    )(Hbt, xa, a, wc_r, smax, xvb, w3, a2)


def _node_pass(Hbt, xa, a2, qmax, y, ymax, eaug):
    n, e, d = N_NODE, N_EDGE, D
    return pl.pallas_call(
        _node_body,
        grid=(e // TE3,),
        in_specs=[_cols((n, TE3)), _full((n, d)), _full((2 * d, 1)),
                  _full((1, 1)), _cols((1, TE3)), _full((1, 1)),
                  _rows((TE3, 2 * d))],
        out_specs=_full((n, 2 * d)),
        out_shape=jax.ShapeDtypeStruct((n, 2 * d), f32),
    )(Hbt, xa, a2, qmax, y, ymax, eaug)


@jax.jit
def kernel(x, H, g1_w2, g1_w3, g1_wc, g1_a, g1_a2,
           g2_w, g2_w2, g2_w3, g2_wc, g2_a, g2_a2):
    n, e, d = N_NODE, N_EDGE, D
    x2 = x[0]
    Ht = H[0].T
    wc1_r = g1_wc.reshape(1, d)
    wc2_r = g2_wc.reshape(1, d)

    Hbt = pl.pallas_call(
        _cast_body,
        grid=(e // TE3,),
        in_specs=[_cols((n, TE3))],
        out_specs=_cols((n, TE3)),
        out_shape=jax.ShapeDtypeStruct((n, e), bf16),
    )(Ht)

    xa1, xvb1, smax1, qmax1 = pl.pallas_call(
        _prep_body,
        grid=(1,),
        in_specs=[_full((n, d)), _full((d, d)), _full((2 * d, 1)),
                  _full((2 * d, 1)), _full((1, d))],
        out_specs=[_full((n, d)), _full((n, 2 * d)), _full((1, 1)),
                   _full((1, 1))],
        out_shape=[jax.ShapeDtypeStruct((n, d), f32),
                   jax.ShapeDtypeStruct((n, 2 * d), bf16),
                   jax.ShapeDtypeStruct((1, 1), f32),
                   jax.ShapeDtypeStruct((1, 1), f32)],
    )(x2, g1_w2, g1_a, g1_a2, wc1_r)

    edge1, eaug1, y1, ymax1 = _edge_pass(Hbt, xa1, g1_a, wc1_r, smax1, xvb1,
                                         g1_w3, g1_a2)
    aug1 = _node_pass(Hbt, xa1, g1_a2, qmax1, y1, ymax1, eaug1)

    xa2, xvb2, smax2, qmax2 = pl.pallas_call(
        _norm_prep_body,
        grid=(1,),
        in_specs=[_full((n, 2 * d)), _full((e, d)), _full((d, d)),
                  _full((d, d)), _full((2 * d, 1)), _full((2 * d, 1)),
                  _full((1, d))],
        out_specs=[_full((n, d)), _full((n, 2 * d)), _full((1, 1)),
                   _full((1, 1))],
        out_shape=[jax.ShapeDtypeStruct((n, d), f32),
                   jax.ShapeDtypeStruct((n, 2 * d), bf16),
                   jax.ShapeDtypeStruct((1, 1), f32),
                   jax.ShapeDtypeStruct((1, 1), f32)],
    )(aug1, edge1, g2_w2, g2_w, g2_a, g2_a2, wc2_r)

    edge2, eaug2, y2, ymax2 = _edge_pass(Hbt, xa2, g2_a, wc2_r, smax2, xvb2,
                                         g2_w3, g2_a2)
    aug2 = _node_pass(Hbt, xa2, g2_a2, qmax2, y2, ymax2, eaug2)

    out = pl.pallas_call(
        _norm_body,
        grid=(1,),
        in_specs=[_full((n, 2 * d)), _full((e, d))],
        out_specs=_full((n, d)),
        out_shape=jax.ShapeDtypeStruct((n, d), f32),
    )(aug2, edge2)
    return out.reshape(1, n, d)


# speedup vs baseline: 1.1865x; 1.1865x over previous
import jax
import jax.numpy as jnp
from jax import lax
from jax.experimental import pallas as pl
from jax.experimental.pallas import tpu as pltpu

ALPHA = 0.2
N_NODE = 10000
N_EDGE = 1024
D = 128

TE = 128
NT = N_EDGE // TE

f32 = jnp.float32
bf16 = jnp.bfloat16


def _lrelu(t):
    return jnp.where(t > 0, t, ALPHA * t)


def _prep(xa, xv, a_ref, a2_ref, wc_ref):
    c = jnp.dot(wc_ref[...], a_ref[0:D, :], preferred_element_type=f32)
    s = _lrelu(c[0, 0] + jnp.dot(xa, a_ref[D:2 * D, :],
                                 preferred_element_type=f32))
    q = jnp.dot(xa, a2_ref[0:D, :], preferred_element_type=f32)
    expw = jnp.exp(s - jnp.max(s))
    qm = jnp.max(q)
    acol = jnp.exp(q - qm).astype(bf16)
    ccol = jnp.exp(ALPHA * q - qm).astype(bf16)
    xvb = jnp.concatenate([xv, jnp.ones((N_NODE, D), f32)], axis=1)
    xvbw = (expw * xvb).astype(bf16)
    mx = jnp.sum(xv, axis=0, keepdims=True) * (1.0 / N_NODE)
    return xvbw, acol, ccol, mx


def _cast_prep_body(ht_ref, x_ref, w2_ref, a_ref, a2_ref, wc_ref,
                    hbt_ref, xvbw_ref, acol_ref, ccol_ref, mx_ref):
    hbt_ref[...] = ht_ref[...].astype(bf16)

    @pl.when(pl.program_id(0) == 0)
    def _():
        x = x_ref[...]
        xa = jnp.dot(x, w2_ref[...], preferred_element_type=f32)
        xvbw, acol, ccol, mx = _prep(xa, x, a_ref, a2_ref, wc_ref)
        xvbw_ref[...] = xvbw
        acol_ref[...] = acol
        ccol_ref[...] = ccol
        mx_ref[...] = mx


def _layer_body(concat, last_out_f32, hbt_ref, xvbw_ref, acol_ref, ccol_ref,
                mx_ref, w3_ref, a2_ref, w2n_ref, wn_ref, an_ref, a2n_ref,
                wcn_ref, o1_ref, o2_ref, o3_ref, o4_ref,
                edge_scr, eaug_scr, y_scr, ymax_scr, aug_scr):
    i = pl.program_id(0)

    @pl.when(i < NT)
    def _edge_phase():
        numaug = lax.dot_general(hbt_ref[...], xvbw_ref[...],
                                 (((0,), (0,)), ((), ())),
                                 preferred_element_type=f32)
        num = numaug[:, :D]
        den = numaug[:, D:D + 1]
        edge = jnp.where(den > 0, num / den, mx_ref[...])
        edge_scr[pl.ds(i * TE, TE), :] = edge
        eaug_scr[pl.ds(i * TE, TE), :] = jnp.concatenate(
            [edge.astype(bf16), jnp.ones((TE, D), bf16)], axis=1)
        w3a = jnp.dot(w3_ref[...], a2_ref[D:2 * D, :],
                      preferred_element_type=f32)
        y = lax.dot_general(w3a, edge, (((0,), (1,)), ((), ())),
                            preferred_element_type=f32)
        y_scr[pl.ds(i, 1), :] = y
        prev = jnp.where(i == 0, jnp.full((1, 1), -jnp.inf, f32),
                         ymax_scr[...])
        ymax_scr[...] = jnp.maximum(prev, jnp.max(y).reshape(1, 1))

    @pl.when((i >= NT) & (i < 2 * NT))
    def _node_phase():
        j = i - NT

        @pl.when(j == 0)
        def _():
            aug_scr[...] = jnp.zeros_like(aug_scr)

        ym = ymax_scr[0, 0]
        y = y_scr[pl.ds(j, 1), :]
        brow = jnp.exp(y - ym).astype(bf16)
        drow = jnp.exp(ALPHA * y - ym).astype(bf16)
        w2 = hbt_ref[...] * jnp.maximum(acol_ref[...] * brow,
                                        ccol_ref[...] * drow)
        aug_scr[...] += lax.dot_general(
            w2, eaug_scr[pl.ds(j * TE, TE), :], (((1,), (0,)), ((), ())),
            preferred_element_type=f32)

    @pl.when(i == 2 * NT)
    def _final():
        aug = aug_scr[...]
        num = aug[:, :D]
        den = aug[:, D:D + 1]
        emean = jnp.sum(edge_scr[...], axis=0, keepdims=True) * (1.0 / N_EDGE)
        node = jnp.where(den > 0, num / den, emean)
        if concat:
            node = jnp.where(node > 0, node, jnp.exp(node) - 1.0)
        if last_out_f32:
            o1_ref[...] = node
        else:
            xa = jnp.dot(node, w2n_ref[...], preferred_element_type=f32)
            xv = jnp.dot(node, wn_ref[...], preferred_element_type=f32)
            xvbw, acol, ccol, mx = _prep(xa, xv, an_ref, a2n_ref, wcn_ref)
            o1_ref[...] = xvbw
            o2_ref[...] = acol
            o3_ref[...] = ccol
            o4_ref[...] = mx


def _full(shape):
    nd = len(shape)
    return pl.BlockSpec(shape, lambda i: (0,) * nd)


def _hbt_spec():
    def idx(i):
        return (0, jnp.where(i < NT, i, jnp.minimum(i - NT, NT - 1)))
    return pl.BlockSpec((N_NODE, TE), idx)


def _layer_call(concat, last, Hbt, xvbw, acol, ccol, mx, w3, a2,
                nxt_params):
    n, e, d = N_NODE, N_EDGE, D
    w2n, wn, an, a2n, wcn = nxt_params
    if last:
        out_shape = [jax.ShapeDtypeStruct((n, d), f32)] * 1 + [
            jax.ShapeDtypeStruct((1, 1), f32)] * 3
        out_specs = [_full((n, d))] + [_full((1, 1))] * 3
    else:
        out_shape = [jax.ShapeDtypeStruct((n, 2 * d), bf16),
                     jax.ShapeDtypeStruct((n, 1), bf16),
                     jax.ShapeDtypeStruct((n, 1), bf16),
                     jax.ShapeDtypeStruct((1, d), f32)]
        out_specs = [_full((n, 2 * d)), _full((n, 1)), _full((n, 1)),
                     _full((1, d))]
    res = pl.pallas_call(
        lambda *refs: _layer_body(concat, last, *refs),
        grid=(2 * NT + 1,),
        in_specs=[_hbt_spec(), _full((n, 2 * d)), _full((n, 1)),
                  _full((n, 1)), _full((1, d)), _full((d, d)),
                  _full((2 * d, 1)), _full((d, d)), _full((d, d)),
                  _full((2 * d, 1)), _full((2 * d, 1)), _full((1, d))],
        out_specs=out_specs,
        out_shape=out_shape,
        scratch_shapes=[
            pltpu.VMEM((e, d), f32),
            pltpu.VMEM((e, 2 * d), bf16),
            pltpu.VMEM((NT, TE), f32),
            pltpu.VMEM((1, 1), f32),
            pltpu.VMEM((n, 2 * d), f32),
        ],
    )(Hbt, xvbw, acol, ccol, mx, w3, a2, w2n, wn, an, a2n, wcn)
    return res


@jax.jit
def kernel(x, H, g1_w2, g1_w3, g1_wc, g1_a, g1_a2,
           g2_w, g2_w2, g2_w3, g2_wc, g2_a, g2_a2):
    n, e, d = N_NODE, N_EDGE, D
    x2 = x[0]
    Ht = H[0].T
    wc1_r = g1_wc.reshape(1, d)
    wc2_r = g2_wc.reshape(1, d)

    Hbt, xvbw1, acol1, ccol1, mx1 = pl.pallas_call(
        _cast_prep_body,
        grid=(NT,),
        in_specs=[pl.BlockSpec((n, TE), lambda i: (0, i)), _full((n, d)),
                  _full((d, d)), _full((2 * d, 1)), _full((2 * d, 1)),
                  _full((1, d))],
        out_specs=[pl.BlockSpec((n, TE), lambda i: (0, i)),
                   _full((n, 2 * d)), _full((n, 1)), _full((n, 1)),
                   _full((1, d))],
        out_shape=[jax.ShapeDtypeStruct((n, e), bf16),
                   jax.ShapeDtypeStruct((n, 2 * d), bf16),
                   jax.ShapeDtypeStruct((n, 1), bf16),
                   jax.ShapeDtypeStruct((n, 1), bf16),
                   jax.ShapeDtypeStruct((1, d), f32)],
    )(Ht, x2, g1_w2, g1_a, g1_a2, wc1_r)

    p2 = (g2_w2, g2_w, g2_a, g2_a2, wc2_r)
    xvbw2, acol2, ccol2, mx2 = _layer_call(
        True, False, Hbt, xvbw1, acol1, ccol1, mx1, g1_w3, g1_a2, p2)
    out, _, _, _ = _layer_call(
        False, True, Hbt, xvbw2, acol2, ccol2, mx2, g2_w3, g2_a2, p2)
    return out.reshape(1, n, d)


# speedup vs baseline: 1.5894x; 1.3396x over previous
import jax
import jax.numpy as jnp
from jax import lax
from jax.experimental import pallas as pl
from jax.experimental.pallas import tpu as pltpu

ALPHA = 0.2
N_NODE = 10000
N_EDGE = 1024
D = 128

TE = 256
NT = N_EDGE // TE
TC_ = 128

f32 = jnp.float32
bf16 = jnp.bfloat16


def _lrelu(t):
    return jnp.where(t > 0, t, ALPHA * t)


def _prep(xa, xv, a_ref, a2_ref, wc_ref):
    c = jnp.dot(wc_ref[...], a_ref[0:D, :], preferred_element_type=f32)
    s = _lrelu(c[0, 0] + jnp.dot(xa, a_ref[D:2 * D, :],
                                 preferred_element_type=f32))
    q = jnp.dot(xa, a2_ref[0:D, :], preferred_element_type=f32)
    expw = jnp.exp(s - jnp.max(s))
    qm = jnp.max(q)
    accol = jnp.concatenate([jnp.exp(q - qm),
                             jnp.exp(ALPHA * q - qm)], axis=1).astype(bf16)
    xvb = jnp.concatenate([xv, jnp.ones((N_NODE, D), f32)], axis=1)
    xvbw = (expw * xvb).astype(bf16)
    mx = jnp.sum(xv, axis=0, keepdims=True) * (1.0 / N_NODE)
    return xvbw, accol, mx


def _cast_prep_body(ht_ref, x_ref, w2_ref, a_ref, a2_ref, wc_ref,
                    hbt_ref, xvbw_ref, accol_ref, mx_ref):
    hbt_ref[...] = ht_ref[...].astype(bf16)

    @pl.when(pl.program_id(0) == 0)
    def _():
        x = x_ref[...]
        xa = jnp.dot(x, w2_ref[...], preferred_element_type=f32)
        xvbw, accol, mx = _prep(xa, x, a_ref, a2_ref, wc_ref)
        xvbw_ref[...] = xvbw
        accol_ref[...] = accol
        mx_ref[...] = mx


def _layer_body(concat, last_out_f32, hbt_ref, xvbw_ref, accol_ref,
                mx_ref, w3_ref, a2_ref, w2n_ref, wn_ref, an_ref, a2n_ref,
                wcn_ref, o1_ref, o2_ref, o3_ref,
                edge_scr, eaug_scr, y_scr, ymax_scr, aug_scr):
    i = pl.program_id(0)

    @pl.when(i < NT)
    def _edge_phase():
        numaug = lax.dot_general(hbt_ref[...], xvbw_ref[...],
                                 (((0,), (0,)), ((), ())),
                                 preferred_element_type=f32)
        num = numaug[:, :D]
        den = numaug[:, D:D + 1]
        edge = jnp.where(den > 0, num / den, mx_ref[...])
        edge_scr[pl.ds(i * TE, TE), :] = edge
        eaug_scr[pl.ds(i * TE, TE), :] = jnp.concatenate(
            [edge.astype(bf16), jnp.ones((TE, D), bf16)], axis=1)
        w3a = jnp.dot(w3_ref[...], a2_ref[D:2 * D, :],
                      preferred_element_type=f32)
        y = lax.dot_general(w3a, edge, (((0,), (1,)), ((), ())),
                            preferred_element_type=f32)
        y_scr[pl.ds(i, 1), :] = y
        prev = jnp.where(i == 0, jnp.full((1, 1), -jnp.inf, f32),
                         ymax_scr[...])
        ymax_scr[...] = jnp.maximum(prev, jnp.max(y).reshape(1, 1))

    @pl.when((i >= NT) & (i < 2 * NT))
    def _node_phase():
        j = i - NT

        @pl.when(j == 0)
        def _():
            aug_scr[...] = jnp.zeros_like(aug_scr)

        ym = ymax_scr[0, 0]
        y = y_scr[pl.ds(j, 1), :]
        brow = jnp.exp(y - ym).astype(bf16)
        drow = jnp.exp(ALPHA * y - ym).astype(bf16)
        w2 = hbt_ref[...] * jnp.maximum(accol_ref[:, 0:1] * brow,
                                        accol_ref[:, 1:2] * drow)
        aug_scr[...] += lax.dot_general(
            w2, eaug_scr[pl.ds(j * TE, TE), :], (((1,), (0,)), ((), ())),
            preferred_element_type=f32)

    @pl.when(i == 2 * NT)
    def _final():
        aug = aug_scr[...]
        num = aug[:, :D]
        den = aug[:, D:D + 1]
        emean = jnp.sum(edge_scr[...], axis=0, keepdims=True) * (1.0 / N_EDGE)
        node = jnp.where(den > 0, num / den, emean)
        if concat:
            node = jnp.where(node > 0, node, jnp.exp(node) - 1.0)
        if last_out_f32:
            o1_ref[...] = node
        else:
            xa = jnp.dot(node, w2n_ref[...], preferred_element_type=f32)
            xv = jnp.dot(node, wn_ref[...], preferred_element_type=f32)
            xvbw, accol, mx = _prep(xa, xv, an_ref, a2n_ref, wcn_ref)
            o1_ref[...] = xvbw
            o2_ref[...] = accol
            o3_ref[...] = mx


def _full(shape):
    nd = len(shape)
    return pl.BlockSpec(shape, lambda i: (0,) * nd)


def _hbt_spec():
    def idx(i):
        return (0, jnp.where(i < NT, i, jnp.minimum(i - NT, NT - 1)))
    return pl.BlockSpec((N_NODE, TE), idx)


def _layer_call(concat, last, Hbt, xvbw, accol, mx, w3, a2,
                nxt_params):
    n, e, d = N_NODE, N_EDGE, D
    w2n, wn, an, a2n, wcn = nxt_params
    if last:
        out_shape = [jax.ShapeDtypeStruct((n, d), f32)] * 1 + [
            jax.ShapeDtypeStruct((1, 1), f32)] * 2
        out_specs = [_full((n, d))] + [_full((1, 1))] * 2
    else:
        out_shape = [jax.ShapeDtypeStruct((n, 2 * d), bf16),
                     jax.ShapeDtypeStruct((n, 2), bf16),
                     jax.ShapeDtypeStruct((1, d), f32)]
        out_specs = [_full((n, 2 * d)), _full((n, 2)), _full((1, d))]
    res = pl.pallas_call(
        lambda *refs: _layer_body(concat, last, *refs),
        grid=(2 * NT + 1,),
        in_specs=[_hbt_spec(), _full((n, 2 * d)), _full((n, 2)),
                  _full((1, d)), _full((d, d)),
                  _full((2 * d, 1)), _full((d, d)), _full((d, d)),
                  _full((2 * d, 1)), _full((2 * d, 1)), _full((1, d))],
        out_specs=out_specs,
        out_shape=out_shape,
        scratch_shapes=[
            pltpu.VMEM((e, d), f32),
            pltpu.VMEM((e, 2 * d), bf16),
            pltpu.VMEM((NT, TE), f32),
            pltpu.VMEM((1, 1), f32),
            pltpu.VMEM((n, 2 * d), f32),
        ],
    )(Hbt, xvbw, accol, mx, w3, a2, w2n, wn, an, a2n, wcn)
    return res


@jax.jit
def kernel(x, H, g1_w2, g1_w3, g1_wc, g1_a, g1_a2,
           g2_w, g2_w2, g2_w3, g2_wc, g2_a, g2_a2):
    n, e, d = N_NODE, N_EDGE, D
    x2 = x[0]
    Ht = H[0].T
    wc1_r = g1_wc.reshape(1, d)
    wc2_r = g2_wc.reshape(1, d)

    Hbt, xvbw1, accol1, mx1 = pl.pallas_call(
        _cast_prep_body,
        grid=(e // TC_,),
        in_specs=[pl.BlockSpec((n, TC_), lambda i: (0, i)), _full((n, d)),
                  _full((d, d)), _full((2 * d, 1)), _full((2 * d, 1)),
                  _full((1, d))],
        out_specs=[pl.BlockSpec((n, TC_), lambda i: (0, i)),
                   _full((n, 2 * d)), _full((n, 2)), _full((1, d))],
        out_shape=[jax.ShapeDtypeStruct((n, e), bf16),
                   jax.ShapeDtypeStruct((n, 2 * d), bf16),
                   jax.ShapeDtypeStruct((n, 2), bf16),
                   jax.ShapeDtypeStruct((1, d), f32)],
    )(Ht, x2, g1_w2, g1_a, g1_a2, wc1_r)

    p2 = (g2_w2, g2_w, g2_a, g2_a2, wc2_r)
    xvbw2, accol2, mx2 = _layer_call(
        True, False, Hbt, xvbw1, accol1, mx1, g1_w3, g1_a2, p2)
    out, _, _ = _layer_call(
        False, True, Hbt, xvbw2, accol2, mx2, g2_w3, g2_a2, p2)
    return out.reshape(1, n, d)
